# baseline (speedup 1.0000x reference)
import functools

import jax
import jax.numpy as jnp
from jax import lax
from jax.experimental import pallas as pl
from jax.experimental.pallas import tpu as pltpu
from jax.experimental.pallas import tpu_sc as plsc

B = 16384
D = 64
HALF = B // 2
MARGIN = 1.0
NENT = 1000000

_info = plsc.get_sparse_core_info()
NC, NS, L = _info.num_cores, _info.num_subcores, _info.num_lanes
NW = NC * NS
PPW = HALF // NW
CH = 128
NCH = (2 * PPW) // CH
GRP = CH // 16

TW = 24192


def _tc_cat_t_body(re_ref, im_ref, out_ref):
    eye = (lax.broadcasted_iota(jnp.int32, (D, D), 0)
           == lax.broadcasted_iota(jnp.int32, (D, D), 1)).astype(jnp.float32)
    dn = (((0,), (0,)), ((), ()))
    out_ref[:, 0:D] = lax.dot_general(
        re_ref[...], eye, dn, preferred_element_type=jnp.float32)
    out_ref[:, D:2 * D] = lax.dot_general(
        im_ref[...], eye, dn, preferred_element_type=jnp.float32)


_tc_cat_t = pl.pallas_call(
    _tc_cat_t_body,
    grid=((NENT + TW - 1) // TW,),
    in_specs=[
        pl.BlockSpec((D, TW), lambda i: (0, i)),
        pl.BlockSpec((D, TW), lambda i: (0, i)),
    ],
    out_specs=pl.BlockSpec((TW, 2 * D), lambda i: (i, 0)),
    out_shape=jax.ShapeDtypeStruct((NENT, 2 * D), jnp.float32),
)


def _sc_body(bh, bt, br, ent, rel,
             score_out, losspart_out,
             idx_h, idx_t, idx_r,
             hrow, trow, rrow,
             score_v, loss_v, isem, gsem):
    w = lax.axis_index("s") * NC + lax.axis_index("c")
    pos_base = w * PPW
    neg_base = HALF + w * PPW

    bases = [pos_base + c * CH if c < NCH // 2 else neg_base + (c - NCH // 2) * CH
             for c in range(NCH)]

    copies = []
    for c in range(NCH):
        copies.append(pltpu.async_copy(bh.at[pl.ds(bases[c], CH)], idx_h.at[c], isem))
        copies.append(pltpu.async_copy(bt.at[pl.ds(bases[c], CH)], idx_t.at[c], isem))
        copies.append(pltpu.async_copy(br.at[pl.ds(bases[c], CH)], idx_r.at[c], isem))
    for cp in copies:
        cp.wait()

    lane = lax.iota(jnp.int32, L)
    shuf = [lane ^ sh for sh in (8, 4, 2, 1)]

    def hsum(v):
        for idx in shuf:
            v = v + v.at[idx].get(mode="promise_in_bounds")
        return v

    for c in range(NCH):
        gathers = [
            pltpu.async_copy(ent.at[idx_h.at[c]], hrow, gsem),
            pltpu.async_copy(ent.at[idx_t.at[c]], trow, gsem),
            pltpu.async_copy(rel.at[idx_r.at[c]], rrow, gsem),
        ]
        for g in gathers:
            g.wait()

        def group_body(g, carry, c=c):
            scores = jnp.zeros((L,), jnp.float32)
            for e in range(16):
                row = g * L + e
                acc = jnp.zeros((L,), jnp.float32)
                for q in range(D // L):
                    re_s = pl.ds(q * L, L)
                    im_s = pl.ds(D + q * L, L)
                    a = hrow[row, re_s]
                    b = hrow[row, im_s]
                    tr = trow[row, re_s]
                    ti = trow[row, im_s]
                    rr = rrow[row, re_s]
                    ri = rrow[row, im_s]
                    acc = acc + (a * tr + b * ti) * rr + (a * ti - b * tr) * ri
                scores = jnp.where(lane == e, hsum(acc), scores)
            score_v[pl.ds(c * CH + g * L, L)] = scores
            return carry

        lax.fori_loop(0, GRP, group_body, 0)

    pltpu.sync_copy(score_v.at[pl.ds(0, PPW)], score_out.at[pl.ds(pos_base, PPW)])
    pltpu.sync_copy(score_v.at[pl.ds(PPW, PPW)], score_out.at[pl.ds(neg_base, PPW)])

    acc = jnp.zeros((L,), jnp.float32)
    for j in range(PPW // L):
        p = score_v[pl.ds(j * L, L)]
        n = score_v[pl.ds(PPW + j * L, L)]
        acc = acc + jnp.maximum(0.0, p - n + MARGIN)
    loss_v[...] = acc
    pltpu.sync_copy(loss_v, losspart_out.at[w])


@functools.partial(
    pl.kernel,
    mesh=plsc.VectorSubcoreMesh(core_axis_name="c", subcore_axis_name="s"),
    compiler_params=pltpu.CompilerParams(use_tc_tiling_on_sc=True),
    out_type=[
        jax.ShapeDtypeStruct((B,), jnp.float32),
        jax.ShapeDtypeStruct((NW, L), jnp.float32),
    ],
    scratch_types=[
        pltpu.VMEM((NCH, CH), jnp.int32),
        pltpu.VMEM((NCH, CH), jnp.int32),
        pltpu.VMEM((NCH, CH), jnp.int32),
        pltpu.VMEM((CH, 2 * D), jnp.float32),
        pltpu.VMEM((CH, 2 * D), jnp.float32),
        pltpu.VMEM((CH, 2 * D), jnp.float32),
        pltpu.VMEM((2 * PPW,), jnp.float32),
        pltpu.VMEM((L,), jnp.float32),
        pltpu.SemaphoreType.DMA,
        pltpu.SemaphoreType.DMA,
    ],
)
def _sc_kernel(bh, bt, br, ent, rel, *rest):
    _sc_body(bh, bt, br, ent, rel, *rest)


def kernel(batch_h, batch_t, batch_r, batch_y, ent_re, ent_im, rel_re, rel_im):
    bh = batch_h.astype(jnp.int32)
    bt = batch_t.astype(jnp.int32)
    br = batch_r.astype(jnp.int32)
    ent = _tc_cat_t(ent_re.T, ent_im.T)
    rel = jnp.concatenate([rel_re, rel_im], axis=1)
    score, losspart = _sc_kernel(bh, bt, br, ent, rel)
    loss = jnp.sum(losspart)
    return (loss, score[:HALF], score[HALF:], score)
